# baseline (speedup 1.0000x reference)
import functools

import jax
import jax.numpy as jnp
from jax import lax
from jax.experimental import pallas as pl
from jax.experimental.pallas import tpu as pltpu
from jax.experimental.pallas import tpu_sc as plsc

B = 64
V = 100000
PADV = 102400
NVR = 25
NBLK = 256
NG = 400
CAP = 128
NEXT = 56
TOPN = 50
NL = 16
NEG_INF = float("-inf")

_LN2 = 0.6931471805599453
_SQRT2 = 1.4142135623730951


def _vlog(v):
    bits = lax.bitcast_convert_type(v, jnp.int32)
    e2 = ((bits >> 23) & 0xFF) - 127
    m = lax.bitcast_convert_type((bits & 0x7FFFFF) | 0x3F800000, jnp.float32)
    big = m > _SQRT2
    m = jnp.where(big, m * 0.5, m)
    e2 = jnp.where(big, e2 + 1, e2)
    z = (m - 1.0) / (m + 1.0)
    z2 = z * z
    poly = 1.0 + z2 * (1.0 / 3.0 + z2 * (0.2 + z2 * (1.0 / 7.0 + z2 * (1.0 / 9.0))))
    return e2.astype(jnp.float32) * _LN2 + 2.0 * z * poly


def _sload(ref, i):
    return plsc.load_gather(ref, [jnp.broadcast_to(i, (NL,)).astype(jnp.int32)])[0]


def _sstore(ref, i, x, lanes):
    plsc.store_scatter(ref, [jnp.broadcast_to(i, (NL,)).astype(jnp.int32)],
                       jnp.broadcast_to(x, (NL,)), mask=lanes == 0)


def _compress_store(ref, off, x, mask):
    mi = mask.astype(jnp.int32)
    cum = plsc.cumsum(mi)
    dest = off + cum - mi
    plsc.store_scatter(ref, [dest], x, mask=mask)
    return off + cum[NL - 1]


def _body(logits_hbm, temp_hbm, topk_hbm, topp_hbm, q_hbm,
          samp_hbm, lp_hbm, idx_hbm,
          rowbuf, gm_ref, glist, cval, cidx, sval, sidx,
          qidx, qval, tbuf, kbuf, pbuf, orow_s, orow_f, orow_i, sem, sem2):
    nc = 2
    wid = lax.axis_index("s") * nc + lax.axis_index("c")

    pltpu.sync_copy(temp_hbm, tbuf.at[pl.ds(0, B)])
    pltpu.sync_copy(topk_hbm, kbuf.at[pl.ds(0, B)])
    pltpu.sync_copy(topp_hbm, pbuf.at[pl.ds(0, B)])

    lanes = lax.iota(jnp.int32, NL)
    ninf = jnp.full((NL,), NEG_INF, jnp.float32)

    def _pad(i, _):
        rowbuf[pl.ds(V + i * NL, NL)] = ninf
        return 0
    lax.fori_loop(0, (PADV - V) // NL, _pad, 0)

    HALF = 51200

    def _per_row(rr, _):
        row = wid * 2 + rr
        c1 = pltpu.async_copy(logits_hbm.at[row, pl.ds(0, HALF)],
                              rowbuf.at[pl.ds(0, HALF)], sem)
        c2 = pltpu.async_copy(logits_hbm.at[row, pl.ds(HALF, V - HALF)],
                              rowbuf.at[pl.ds(HALF, V - HALF)], sem2)

        def _ga(b, accs):
            base = b * (NVR * NL)
            return tuple(
                jnp.maximum(accs[v], rowbuf[pl.ds(base + v * NL, NL)])
                for v in range(NVR)
            )
        c1.wait()
        accs = lax.fori_loop(0, NBLK // 2, _ga, tuple(ninf for _ in range(NVR)))
        c2.wait()
        accs = lax.fori_loop(NBLK // 2, NBLK, _ga, accs)
        for v in range(NVR):
            gm_ref[pl.ds(v * NL, NL)] = accs[v]

        gmaxv = accs[0]
        gminv = accs[0]
        for v in range(1, NVR):
            gmaxv = jnp.maximum(gmaxv, accs[v])
            gminv = jnp.minimum(gminv, accs[v])
        hi0 = jnp.max(gmaxv)
        lo0 = jnp.min(gminv)

        def _bs(_, carry):
            lo, hi, best = carry
            mid = 0.5 * (lo + hi)
            cnt = jnp.zeros((NL,), jnp.int32)
            for v in range(NVR):
                gv = gm_ref[pl.ds(v * NL, NL)]
                cnt = cnt + plsc.all_reduce_population_count(gv >= mid)
            ok = cnt[0] >= TOPN
            return (jnp.where(ok, mid, lo),
                    jnp.where(ok, hi, mid),
                    jnp.where(ok, mid, best))
        _, _, t = lax.fori_loop(0, 20, _bs, (lo0, hi0, lo0))

        if False:
            orow_s[...] = lanes + t.astype(jnp.int32)
            pltpu.sync_copy(orow_s, samp_hbm.at[row])
            pltpu.sync_copy(orow_s, idx_hbm.at[row])
            pltpu.sync_copy(gm_ref.at[pl.ds(0, NL)], lp_hbm.at[row])
            return 0

        goff = jnp.int32(0)
        for v in range(NVR):
            gv = gm_ref[pl.ds(v * NL, NL)]
            goff = _compress_store(glist, goff, lanes + v * NL, gv >= t)
        ngr = goff

        for v in range(CAP // NL + 1):
            cval[pl.ds(v * NL, NL)] = ninf

        def _gather_grp(j, cl):
            g = _sload(glist, j)
            for jb in range(NBLK // NL):
                pos = (lanes + jb * NL) * (NVR * NL) + g
                vals = plsc.load_gather(rowbuf, [pos])
                cmask = vals >= t
                _compress_store(cval, cl, vals, cmask)
                cl = _compress_store(cidx, cl, pos, cmask)
                cl = jnp.minimum(cl, CAP - NL)
            return cl
        lax.fori_loop(0, ngr, _gather_grp, jnp.int32(0))


        for v in range(4):
            sval[pl.ds(v * NL, NL)] = ninf
            sidx[pl.ds(v * NL, NL)] = jnp.zeros((NL,), jnp.int32)

        big = jnp.full((NL,), jnp.int32(0x7FFFFFF), jnp.int32)

        def _ext(e, _):
            cvs = [cval[pl.ds(v * NL, NL)] for v in range(CAP // NL)]
            ivs = [cidx[pl.ds(v * NL, NL)] for v in range(CAP // NL)]
            mv = cvs[0]
            for v in range(1, CAP // NL):
                mv = jnp.maximum(mv, cvs[v])
            m = jnp.max(mv)
            tv = jnp.full((NL,), -1, jnp.int32)
            for v in range(CAP // NL):
                tv = jnp.maximum(tv, jnp.where(cvs[v] == m, ivs[v], -1))
            tgt = jnp.max(tv)
            pv = big
            for v in range(CAP // NL):
                pv = jnp.minimum(pv, jnp.where((cvs[v] == m) & (ivs[v] == tgt),
                                               lanes + v * NL, big))
            pos = jnp.min(pv)
            _sstore(sval, e, m, lanes)
            _sstore(sidx, e, tgt, lanes)
            _sstore(cval, pos, NEG_INF, lanes)
            return 0
        lax.fori_loop(0, NEXT, _ext, 0)

        temp = _sload(tbuf, row)
        k = _sload(kbuf, row)
        p = _sload(pbuf, row)
        thresh = _sload(sval, k - 1)
        tempv = jnp.broadcast_to(temp, (NL,))

        xv = [sval[pl.ds(v * NL, NL)] / tempv for v in range(4)]
        m_x = xv[0][0]
        km = [sval[pl.ds(v * NL, NL)] >= thresh for v in range(4)]
        ev = [jnp.where(km[v], jnp.exp(jnp.where(km[v], xv[v] - m_x, 0.0)), 0.0)
              for v in range(4)]
        s1 = jnp.sum(ev[0] + ev[1] + ev[2] + ev[3])

        carry = jnp.float32(0.0)
        cex = []
        for v in range(4):
            c_in = plsc.cumsum(ev[v])
            cex.append(c_in - ev[v] + carry)
            carry = carry + jnp.sum(ev[v])
        pt = p * s1
        keep = [(cex[v] < pt) & km[v] for v in range(4)]


        s2 = jnp.float32(0.0)
        for v in range(4):
            s2 = s2 + jnp.sum(jnp.where(keep[v], ev[v], 0.0))
        logs2 = _vlog(jnp.full((NL,), s2, jnp.float32))[0]

        for v in range(4):
            iv = sidx[pl.ds(v * NL, NL)]
            qidx[pl.ds(v * NL, NL)] = jnp.where(keep[v], iv, lanes + v * NL)
        cq = pltpu.async_copy(q_hbm.at[row].at[qidx], qval, sem)

        orow_f[...] = jnp.zeros((NL,), jnp.float32)
        orow_i[...] = jnp.zeros((NL,), jnp.int32)
        wv = [jnp.where(keep[v], sval[pl.ds(v * NL, NL)], NEG_INF)
              for v in range(4)]
        si = [sidx[pl.ds(v * NL, NL)] for v in range(4)]
        for j in range(5):
            mj = jnp.max(jnp.maximum(jnp.maximum(wv[0], wv[1]),
                                     jnp.maximum(wv[2], wv[3])))
            tj = big
            for v in range(4):
                tj = jnp.minimum(tj, jnp.where(wv[v] == mj, si[v], big))
            tgt5 = jnp.min(tj)
            lp_j = (jnp.broadcast_to(mj, (NL,)) / tempv)[0] - m_x - logs2
            _sstore(orow_f, j, lp_j, lanes)
            _sstore(orow_i, j, tgt5, lanes)
            for v in range(4):
                wv[v] = jnp.where((wv[v] == mj) & (si[v] == tgt5),
                                  ninf, wv[v])
        cq.wait()

        rmaxv = jnp.full((NL,), NEG_INF, jnp.float32)
        ratios = []
        for v in range(4):
            qv = jnp.minimum(jnp.maximum(qval[pl.ds(v * NL, NL)], 1e-10), 1.0)
            expo = -_vlog(qv)
            r = jnp.where(keep[v], (ev[v] / s2) / expo, -1.0)
            ratios.append(r)
            rmaxv = jnp.maximum(rmaxv, r)
        rmax = jnp.max(rmaxv)
        pv = big
        for v in range(4):
            pv = jnp.minimum(pv, jnp.where(ratios[v] == rmax, lanes + v * NL, big))
        spos = jnp.min(pv)
        sampled = _sload(sidx, spos)
        lp_samp = (jnp.broadcast_to(_sload(sval, spos), (NL,)) / tempv)[0] - m_x - logs2

        _sstore(orow_f, 5, lp_samp, lanes)
        _sstore(orow_i, 5, sampled, lanes)
        orow_s[...] = jnp.where(lanes == 0, sampled, 0)
        pltpu.sync_copy(orow_s, samp_hbm.at[row])
        pltpu.sync_copy(orow_f, lp_hbm.at[row])
        pltpu.sync_copy(orow_i, idx_hbm.at[row])
        return 0

    lax.fori_loop(0, 2, _per_row, 0)


@jax.jit
def _sc_sampler(logits, temperature, top_k, top_p, q):
    mesh = plsc.VectorSubcoreMesh(core_axis_name="c", subcore_axis_name="s")
    f = pl.kernel(
        _body,
        out_type=[
            jax.ShapeDtypeStruct((B, NL), jnp.int32),
            jax.ShapeDtypeStruct((B, NL), jnp.float32),
            jax.ShapeDtypeStruct((B, NL), jnp.int32),
        ],
        mesh=mesh,
        compiler_params=pltpu.CompilerParams(needs_layout_passes=False,
                                             use_tc_tiling_on_sc=False),
        scratch_types=[
            pltpu.VMEM((PADV,), jnp.float32),
            pltpu.VMEM((NG,), jnp.float32),
            pltpu.VMEM((NG + 32,), jnp.int32),
            pltpu.VMEM((CAP + NL,), jnp.float32),
            pltpu.VMEM((CAP + NL,), jnp.int32),
            pltpu.VMEM((64 + NL,), jnp.float32),
            pltpu.VMEM((64 + NL,), jnp.int32),
            pltpu.VMEM((64,), jnp.int32),
            pltpu.VMEM((64,), jnp.float32),
            pltpu.VMEM((B + NL,), jnp.float32),
            pltpu.VMEM((B + NL,), jnp.int32),
            pltpu.VMEM((B + NL,), jnp.float32),
            pltpu.VMEM((NL,), jnp.int32),
            pltpu.VMEM((NL,), jnp.float32),
            pltpu.VMEM((NL,), jnp.int32),
            pltpu.SemaphoreType.DMA,
            pltpu.SemaphoreType.DMA,
        ],
    )
    return f(logits, temperature, top_k, top_p, q)


def kernel(logits, temperature, top_k, top_p, q):
    samp, lp, idx = _sc_sampler(logits, temperature, top_k, top_p, q)
    return samp[:, 0], lp[:, :6], idx[:, :6]


# speedup vs baseline: 1.0039x; 1.0039x over previous
import functools

import jax
import jax.numpy as jnp
from jax import lax
from jax.experimental import pallas as pl
from jax.experimental.pallas import tpu as pltpu
from jax.experimental.pallas import tpu_sc as plsc

B = 64
V = 100000
PADV = 102400
NVR = 25
NBLK = 256
NG = 400
CAP = 128
NEXT = 56
TOPN = 50
NL = 16
NEG_INF = float("-inf")

_LN2 = 0.6931471805599453
_SQRT2 = 1.4142135623730951


def _vlog(v):
    bits = lax.bitcast_convert_type(v, jnp.int32)
    e2 = ((bits >> 23) & 0xFF) - 127
    m = lax.bitcast_convert_type((bits & 0x7FFFFF) | 0x3F800000, jnp.float32)
    big = m > _SQRT2
    m = jnp.where(big, m * 0.5, m)
    e2 = jnp.where(big, e2 + 1, e2)
    z = (m - 1.0) / (m + 1.0)
    z2 = z * z
    poly = 1.0 + z2 * (1.0 / 3.0 + z2 * (0.2 + z2 * (1.0 / 7.0 + z2 * (1.0 / 9.0))))
    return e2.astype(jnp.float32) * _LN2 + 2.0 * z * poly


def _sload(ref, i):
    return plsc.load_gather(ref, [jnp.broadcast_to(i, (NL,)).astype(jnp.int32)])[0]


def _sstore(ref, i, x, lanes):
    plsc.store_scatter(ref, [jnp.broadcast_to(i, (NL,)).astype(jnp.int32)],
                       jnp.broadcast_to(x, (NL,)), mask=lanes == 0)


def _compress_store(ref, off, x, mask):
    mi = mask.astype(jnp.int32)
    cum = plsc.cumsum(mi)
    dest = off + cum - mi
    plsc.store_scatter(ref, [dest], x, mask=mask)
    return off + cum[NL - 1]


def _body(logits_hbm, temp_hbm, topk_hbm, topp_hbm, q_hbm,
          samp_hbm, lp_hbm, idx_hbm,
          rowbuf, gm_ref, glist, cval, cidx, sval, sidx,
          qidx, qval, tbuf, kbuf, pbuf, orow_s, orow_f, orow_i, sem):
    nc = 2
    wid = lax.axis_index("s") * nc + lax.axis_index("c")

    pltpu.sync_copy(temp_hbm, tbuf.at[pl.ds(0, B)])
    pltpu.sync_copy(topk_hbm, kbuf.at[pl.ds(0, B)])
    pltpu.sync_copy(topp_hbm, pbuf.at[pl.ds(0, B)])

    lanes = lax.iota(jnp.int32, NL)
    ninf = jnp.full((NL,), NEG_INF, jnp.float32)

    def _pad(i, _):
        rowbuf[pl.ds(V + i * NL, NL)] = ninf
        return 0
    lax.fori_loop(0, (PADV - V) // NL, _pad, 0)

    def _per_row(rr, _):
        row = wid * 2 + rr
        pltpu.sync_copy(logits_hbm.at[row], rowbuf.at[pl.ds(0, V)])

        def _ga(b, accs):
            base = b * (NVR * NL)
            return tuple(
                jnp.maximum(accs[v], rowbuf[pl.ds(base + v * NL, NL)])
                for v in range(NVR)
            )
        accs = lax.fori_loop(0, NBLK, _ga, tuple(ninf for _ in range(NVR)))
        for v in range(NVR):
            gm_ref[pl.ds(v * NL, NL)] = accs[v]

        gmaxv = accs[0]
        gminv = accs[0]
        for v in range(1, NVR):
            gmaxv = jnp.maximum(gmaxv, accs[v])
            gminv = jnp.minimum(gminv, accs[v])
        hi0 = jnp.max(gmaxv)
        lo0 = jnp.min(gminv)

        def _bs(_, carry):
            lo, hi, best = carry
            mid = 0.5 * (lo + hi)
            cnt = jnp.zeros((NL,), jnp.int32)
            for v in range(NVR):
                gv = gm_ref[pl.ds(v * NL, NL)]
                cnt = cnt + plsc.all_reduce_population_count(gv >= mid)
            ok = cnt[0] >= TOPN
            return (jnp.where(ok, mid, lo),
                    jnp.where(ok, hi, mid),
                    jnp.where(ok, mid, best))
        _, _, t = lax.fori_loop(0, 20, _bs, (lo0, hi0, lo0))

        if False:
            orow_s[...] = lanes + t.astype(jnp.int32)
            pltpu.sync_copy(orow_s, samp_hbm.at[row])
            pltpu.sync_copy(orow_s, idx_hbm.at[row])
            pltpu.sync_copy(gm_ref.at[pl.ds(0, NL)], lp_hbm.at[row])
            return 0

        goff = jnp.int32(0)
        for v in range(NVR):
            gv = gm_ref[pl.ds(v * NL, NL)]
            goff = _compress_store(glist, goff, lanes + v * NL, gv >= t)
        ngr = goff

        for v in range(CAP // NL + 1):
            cval[pl.ds(v * NL, NL)] = ninf

        def _gather_grp(j, cl):
            g = _sload(glist, j)
            for jb in range(NBLK // NL):
                pos = (lanes + jb * NL) * (NVR * NL) + g
                vals = plsc.load_gather(rowbuf, [pos])
                cmask = vals >= t
                _compress_store(cval, cl, vals, cmask)
                cl = _compress_store(cidx, cl, pos, cmask)
                cl = jnp.minimum(cl, CAP - NL)
            return cl
        lax.fori_loop(0, ngr, _gather_grp, jnp.int32(0))


        for v in range(4):
            sval[pl.ds(v * NL, NL)] = ninf
            sidx[pl.ds(v * NL, NL)] = jnp.zeros((NL,), jnp.int32)

        big = jnp.full((NL,), jnp.int32(0x7FFFFFF), jnp.int32)

        def _ext(e, _):
            cvs = [cval[pl.ds(v * NL, NL)] for v in range(CAP // NL)]
            ivs = [cidx[pl.ds(v * NL, NL)] for v in range(CAP // NL)]
            mv = cvs[0]
            for v in range(1, CAP // NL):
                mv = jnp.maximum(mv, cvs[v])
            m = jnp.max(mv)
            tv = jnp.full((NL,), -1, jnp.int32)
            for v in range(CAP // NL):
                tv = jnp.maximum(tv, jnp.where(cvs[v] == m, ivs[v], -1))
            tgt = jnp.max(tv)
            pv = big
            for v in range(CAP // NL):
                pv = jnp.minimum(pv, jnp.where((cvs[v] == m) & (ivs[v] == tgt),
                                               lanes + v * NL, big))
            pos = jnp.min(pv)
            _sstore(sval, e, m, lanes)
            _sstore(sidx, e, tgt, lanes)
            _sstore(cval, pos, NEG_INF, lanes)
            return 0
        lax.fori_loop(0, NEXT, _ext, 0)

        temp = _sload(tbuf, row)
        k = _sload(kbuf, row)
        p = _sload(pbuf, row)
        thresh = _sload(sval, k - 1)
        tempv = jnp.broadcast_to(temp, (NL,))

        xv = [sval[pl.ds(v * NL, NL)] / tempv for v in range(4)]
        m_x = xv[0][0]
        km = [sval[pl.ds(v * NL, NL)] >= thresh for v in range(4)]
        ev = [jnp.where(km[v], jnp.exp(jnp.where(km[v], xv[v] - m_x, 0.0)), 0.0)
              for v in range(4)]
        s1 = jnp.sum(ev[0] + ev[1] + ev[2] + ev[3])

        carry = jnp.float32(0.0)
        cex = []
        for v in range(4):
            c_in = plsc.cumsum(ev[v])
            cex.append(c_in - ev[v] + carry)
            carry = carry + jnp.sum(ev[v])
        pt = p * s1
        keep = [(cex[v] < pt) & km[v] for v in range(4)]


        s2 = jnp.float32(0.0)
        for v in range(4):
            s2 = s2 + jnp.sum(jnp.where(keep[v], ev[v], 0.0))
        logs2 = _vlog(jnp.full((NL,), s2, jnp.float32))[0]

        for v in range(4):
            iv = sidx[pl.ds(v * NL, NL)]
            qidx[pl.ds(v * NL, NL)] = jnp.where(keep[v], iv, lanes + v * NL)
        pltpu.async_copy(q_hbm.at[row].at[qidx], qval, sem).wait()

        rmaxv = jnp.full((NL,), NEG_INF, jnp.float32)
        ratios = []
        for v in range(4):
            qv = jnp.minimum(jnp.maximum(qval[pl.ds(v * NL, NL)], 1e-10), 1.0)
            expo = -_vlog(qv)
            r = jnp.where(keep[v], (ev[v] / s2) / expo, -1.0)
            ratios.append(r)
            rmaxv = jnp.maximum(rmaxv, r)
        rmax = jnp.max(rmaxv)
        pv = big
        for v in range(4):
            pv = jnp.minimum(pv, jnp.where(ratios[v] == rmax, lanes + v * NL, big))
        spos = jnp.min(pv)
        sampled = _sload(sidx, spos)
        lp_samp = (jnp.broadcast_to(_sload(sval, spos), (NL,)) / tempv)[0] - m_x - logs2

        orow_f[...] = jnp.where(lanes == 5, lp_samp, 0.0)
        orow_i[...] = jnp.where(lanes == 5, sampled, 0)
        wv = [jnp.where(keep[v], sval[pl.ds(v * NL, NL)], NEG_INF)
              for v in range(4)]
        si = [sidx[pl.ds(v * NL, NL)] for v in range(4)]
        for j in range(5):
            mj = jnp.max(jnp.maximum(jnp.maximum(wv[0], wv[1]),
                                     jnp.maximum(wv[2], wv[3])))
            tj = big
            for v in range(4):
                tj = jnp.minimum(tj, jnp.where(wv[v] == mj, si[v], big))
            tgt5 = jnp.min(tj)
            lp_j = (jnp.broadcast_to(mj, (NL,)) / tempv)[0] - m_x - logs2
            _sstore(orow_f, j, lp_j, lanes)
            _sstore(orow_i, j, tgt5, lanes)
            for v in range(4):
                wv[v] = jnp.where((wv[v] == mj) & (si[v] == tgt5),
                                  ninf, wv[v])
        orow_s[...] = jnp.where(lanes == 0, sampled, 0)
        pltpu.sync_copy(orow_s, samp_hbm.at[row])
        pltpu.sync_copy(orow_f, lp_hbm.at[row])
        pltpu.sync_copy(orow_i, idx_hbm.at[row])
        return 0

    lax.fori_loop(0, 2, _per_row, 0)


@jax.jit
def _sc_sampler(logits, temperature, top_k, top_p, q):
    mesh = plsc.VectorSubcoreMesh(core_axis_name="c", subcore_axis_name="s")
    f = pl.kernel(
        _body,
        out_type=[
            jax.ShapeDtypeStruct((B, NL), jnp.int32),
            jax.ShapeDtypeStruct((B, NL), jnp.float32),
            jax.ShapeDtypeStruct((B, NL), jnp.int32),
        ],
        mesh=mesh,
        compiler_params=pltpu.CompilerParams(needs_layout_passes=False,
                                             use_tc_tiling_on_sc=False),
        scratch_types=[
            pltpu.VMEM((PADV,), jnp.float32),
            pltpu.VMEM((NG,), jnp.float32),
            pltpu.VMEM((NG + 32,), jnp.int32),
            pltpu.VMEM((CAP + NL,), jnp.float32),
            pltpu.VMEM((CAP + NL,), jnp.int32),
            pltpu.VMEM((64 + NL,), jnp.float32),
            pltpu.VMEM((64 + NL,), jnp.int32),
            pltpu.VMEM((64,), jnp.int32),
            pltpu.VMEM((64,), jnp.float32),
            pltpu.VMEM((B + NL,), jnp.float32),
            pltpu.VMEM((B + NL,), jnp.int32),
            pltpu.VMEM((B + NL,), jnp.float32),
            pltpu.VMEM((NL,), jnp.int32),
            pltpu.VMEM((NL,), jnp.float32),
            pltpu.VMEM((NL,), jnp.int32),
            pltpu.SemaphoreType.DMA,
        ],
    )
    return f(logits, temperature, top_k, top_p, q)


def kernel(logits, temperature, top_k, top_p, q):
    samp, lp, idx = _sc_sampler(logits, temperature, top_k, top_p, q)
    return samp[:, 0], lp[:, :6], idx[:, :6]


# speedup vs baseline: 1.0059x; 1.0019x over previous
import jax
import jax.numpy as jnp
from jax import lax
from jax.experimental import pallas as pl
from jax.experimental.pallas import tpu as pltpu
from jax.experimental.pallas import tpu_sc as plsc

B = 64
V = 100000
PADV = 102400
NVR = 25
NBLK = 256
NG = 400
CAP = 128
NEXT = 56
TOPN = 50
NL = 16
NEG_INF = float("-inf")

_LN2 = 0.6931471805599453
_SQRT2 = 1.4142135623730951


def _vlog(v):
    bits = lax.bitcast_convert_type(v, jnp.int32)
    e2 = ((bits >> 23) & 0xFF) - 127
    m = lax.bitcast_convert_type((bits & 0x7FFFFF) | 0x3F800000, jnp.float32)
    big = m > _SQRT2
    m = jnp.where(big, m * 0.5, m)
    e2 = jnp.where(big, e2 + 1, e2)
    z = (m - 1.0) / (m + 1.0)
    z2 = z * z
    poly = 1.0 + z2 * (1.0 / 3.0 + z2 * (0.2 + z2 * (1.0 / 7.0 + z2 * (1.0 / 9.0))))
    return e2.astype(jnp.float32) * _LN2 + 2.0 * z * poly


def _sload(ref, i):
    return plsc.load_gather(ref, [jnp.broadcast_to(i, (NL,)).astype(jnp.int32)])[0]


def _sstore(ref, i, x, lanes):
    plsc.store_scatter(ref, [jnp.broadcast_to(i, (NL,)).astype(jnp.int32)],
                       jnp.broadcast_to(x, (NL,)), mask=lanes == 0)


def _compress_store(ref, off, x, mask):
    mi = mask.astype(jnp.int32)
    cum = plsc.cumsum(mi)
    dest = off + cum - mi
    plsc.store_scatter(ref, [dest], x, mask=mask)
    return off + cum[NL - 1]


def _body(logits_hbm, temp_hbm, topk_hbm, topp_hbm, q_hbm,
          samp_hbm, lp_hbm, idx_hbm,
          rowbuf, gm_ref, glist, cval, cidx, sval, sidx,
          qidx, qval, tbuf, kbuf, pbuf, orow_s, orow_f, orow_i, sem):
    nc = 2
    wid = lax.axis_index("s") * nc + lax.axis_index("c")

    pltpu.sync_copy(temp_hbm, tbuf.at[pl.ds(0, B)])
    pltpu.sync_copy(topk_hbm, kbuf.at[pl.ds(0, B)])
    pltpu.sync_copy(topp_hbm, pbuf.at[pl.ds(0, B)])

    lanes = lax.iota(jnp.int32, NL)
    ninf = jnp.full((NL,), NEG_INF, jnp.float32)

    def _pad(i, _):
        rowbuf[pl.ds(V + i * NL, NL)] = ninf
        return 0
    lax.fori_loop(0, (PADV - V) // NL, _pad, 0)

    def _per_row(rr, _):
        row = wid * 2 + rr
        pltpu.sync_copy(logits_hbm.at[row], rowbuf.at[pl.ds(0, V)])

        def _ga(b, accs):
            base = b * (NVR * NL)
            return tuple(
                jnp.maximum(accs[v], rowbuf[pl.ds(base + v * NL, NL)])
                for v in range(NVR)
            )
        accs = lax.fori_loop(0, NBLK, _ga, tuple(ninf for _ in range(NVR)))
        for v in range(NVR):
            gm_ref[pl.ds(v * NL, NL)] = accs[v]

        gmaxv = accs[0]
        gminv = accs[0]
        for v in range(1, NVR):
            gmaxv = jnp.maximum(gmaxv, accs[v])
            gminv = jnp.minimum(gminv, accs[v])
        hi0 = jnp.max(gmaxv)
        lo0 = jnp.min(gminv)

        def _bs(_, carry):
            lo, hi, best = carry
            mid = 0.5 * (lo + hi)
            cnt = jnp.zeros((NL,), jnp.int32)
            for v in range(NVR):
                gv = gm_ref[pl.ds(v * NL, NL)]
                cnt = cnt + plsc.all_reduce_population_count(gv >= mid)
            ok = cnt[0] >= TOPN
            return (jnp.where(ok, mid, lo),
                    jnp.where(ok, hi, mid),
                    jnp.where(ok, mid, best))
        _, _, t = lax.fori_loop(0, 20, _bs, (lo0, hi0, lo0))

        goff = jnp.int32(0)
        for v in range(NVR):
            gv = gm_ref[pl.ds(v * NL, NL)]
            goff = _compress_store(glist, goff, lanes + v * NL, gv >= t)
        ngr = goff

        for v in range(CAP // NL + 1):
            cval[pl.ds(v * NL, NL)] = ninf

        def _gather_grp(j, cl):
            g = _sload(glist, j)
            for jb in range(NBLK // NL):
                pos = (lanes + jb * NL) * (NVR * NL) + g
                vals = plsc.load_gather(rowbuf, [pos])
                cmask = vals >= t
                _compress_store(cval, cl, vals, cmask)
                cl = _compress_store(cidx, cl, pos, cmask)
                cl = jnp.minimum(cl, CAP - NL)
            return cl
        lax.fori_loop(0, ngr, _gather_grp, jnp.int32(0))


        for v in range(4):
            sval[pl.ds(v * NL, NL)] = ninf
            sidx[pl.ds(v * NL, NL)] = jnp.zeros((NL,), jnp.int32)

        big = jnp.full((NL,), jnp.int32(0x7FFFFFF), jnp.int32)

        def _ext(e, _):
            cvs = [cval[pl.ds(v * NL, NL)] for v in range(CAP // NL)]
            ivs = [cidx[pl.ds(v * NL, NL)] for v in range(CAP // NL)]
            mv = cvs[0]
            for v in range(1, CAP // NL):
                mv = jnp.maximum(mv, cvs[v])
            m = jnp.max(mv)
            tv = jnp.full((NL,), -1, jnp.int32)
            for v in range(CAP // NL):
                tv = jnp.maximum(tv, jnp.where(cvs[v] == m, ivs[v], -1))
            tgt = jnp.max(tv)
            pv = big
            for v in range(CAP // NL):
                pv = jnp.minimum(pv, jnp.where((cvs[v] == m) & (ivs[v] == tgt),
                                               lanes + v * NL, big))
            pos = jnp.min(pv)
            _sstore(sval, e, m, lanes)
            _sstore(sidx, e, tgt, lanes)
            _sstore(cval, pos, NEG_INF, lanes)
            return 0
        lax.fori_loop(0, NEXT, _ext, 0)

        temp = _sload(tbuf, row)
        k = _sload(kbuf, row)
        p = _sload(pbuf, row)
        thresh = _sload(sval, k - 1)
        tempv = jnp.broadcast_to(temp, (NL,))

        xv = [sval[pl.ds(v * NL, NL)] / tempv for v in range(4)]
        m_x = xv[0][0]
        km = [sval[pl.ds(v * NL, NL)] >= thresh for v in range(4)]
        ev = [jnp.where(km[v], jnp.exp(jnp.where(km[v], xv[v] - m_x, 0.0)), 0.0)
              for v in range(4)]
        s1 = jnp.sum(ev[0] + ev[1] + ev[2] + ev[3])

        carry = jnp.float32(0.0)
        cex = []
        for v in range(4):
            c_in = plsc.cumsum(ev[v])
            cex.append(c_in - ev[v] + carry)
            carry = carry + jnp.sum(ev[v])
        pt = p * s1
        keep = [(cex[v] < pt) & km[v] for v in range(4)]


        s2 = jnp.float32(0.0)
        for v in range(4):
            s2 = s2 + jnp.sum(jnp.where(keep[v], ev[v], 0.0))
        logs2 = _vlog(jnp.full((NL,), s2, jnp.float32))[0]

        for v in range(4):
            iv = sidx[pl.ds(v * NL, NL)]
            qidx[pl.ds(v * NL, NL)] = jnp.where(keep[v], iv, lanes + v * NL)
        pltpu.async_copy(q_hbm.at[row].at[qidx], qval, sem).wait()

        rmaxv = jnp.full((NL,), NEG_INF, jnp.float32)
        ratios = []
        for v in range(4):
            qv = jnp.minimum(jnp.maximum(qval[pl.ds(v * NL, NL)], 1e-10), 1.0)
            expo = -_vlog(qv)
            r = jnp.where(keep[v], (ev[v] / s2) / expo, -1.0)
            ratios.append(r)
            rmaxv = jnp.maximum(rmaxv, r)
        rmax = jnp.max(rmaxv)
        pv = big
        for v in range(4):
            pv = jnp.minimum(pv, jnp.where(ratios[v] == rmax, lanes + v * NL, big))
        spos = jnp.min(pv)
        sampled = _sload(sidx, spos)
        lp_samp = (jnp.broadcast_to(_sload(sval, spos), (NL,)) / tempv)[0] - m_x - logs2

        orow_f[...] = jnp.where(lanes == 5, lp_samp, 0.0)
        orow_i[...] = jnp.where(lanes == 5, sampled, 0)
        wv = [jnp.where(keep[v], sval[pl.ds(v * NL, NL)], NEG_INF)
              for v in range(4)]
        si = [sidx[pl.ds(v * NL, NL)] for v in range(4)]
        for j in range(5):
            mj = jnp.max(jnp.maximum(jnp.maximum(wv[0], wv[1]),
                                     jnp.maximum(wv[2], wv[3])))
            tj = big
            for v in range(4):
                tj = jnp.minimum(tj, jnp.where(wv[v] == mj, si[v], big))
            tgt5 = jnp.min(tj)
            lp_j = (jnp.broadcast_to(mj, (NL,)) / tempv)[0] - m_x - logs2
            _sstore(orow_f, j, lp_j, lanes)
            _sstore(orow_i, j, tgt5, lanes)
            for v in range(4):
                wv[v] = jnp.where((wv[v] == mj) & (si[v] == tgt5),
                                  ninf, wv[v])
        orow_s[...] = jnp.where(lanes == 0, sampled, 0)
        pltpu.sync_copy(orow_s, samp_hbm.at[row])
        pltpu.sync_copy(orow_f, lp_hbm.at[row])
        pltpu.sync_copy(orow_i, idx_hbm.at[row])
        return 0

    lax.fori_loop(0, 2, _per_row, 0)


@jax.jit
def _sc_sampler(logits, temperature, top_k, top_p, q):
    mesh = plsc.VectorSubcoreMesh(core_axis_name="c", subcore_axis_name="s")
    f = pl.kernel(
        _body,
        out_type=[
            jax.ShapeDtypeStruct((B, NL), jnp.int32),
            jax.ShapeDtypeStruct((B, NL), jnp.float32),
            jax.ShapeDtypeStruct((B, NL), jnp.int32),
        ],
        mesh=mesh,
        compiler_params=pltpu.CompilerParams(needs_layout_passes=False,
                                             use_tc_tiling_on_sc=False),
        scratch_types=[
            pltpu.VMEM((PADV,), jnp.float32),
            pltpu.VMEM((NG,), jnp.float32),
            pltpu.VMEM((NG + 32,), jnp.int32),
            pltpu.VMEM((CAP + NL,), jnp.float32),
            pltpu.VMEM((CAP + NL,), jnp.int32),
            pltpu.VMEM((64 + NL,), jnp.float32),
            pltpu.VMEM((64 + NL,), jnp.int32),
            pltpu.VMEM((64,), jnp.int32),
            pltpu.VMEM((64,), jnp.float32),
            pltpu.VMEM((B + NL,), jnp.float32),
            pltpu.VMEM((B + NL,), jnp.int32),
            pltpu.VMEM((B + NL,), jnp.float32),
            pltpu.VMEM((NL,), jnp.int32),
            pltpu.VMEM((NL,), jnp.float32),
            pltpu.VMEM((NL,), jnp.int32),
            pltpu.SemaphoreType.DMA,
        ],
    )
    return f(logits, temperature, top_k, top_p, q)


def kernel(logits, temperature, top_k, top_p, q):
    samp, lp, idx = _sc_sampler(logits, temperature, top_k, top_p, q)
    return samp[:, 0], lp[:, :6], idx[:, :6]
